# baseline (speedup 1.0000x reference)
import functools

import jax
import jax.numpy as jnp
from jax import lax
from jax.experimental import pallas as pl
from jax.experimental.pallas import tpu as pltpu
from jax.experimental.pallas import tpu_sc as plsc

B = 16384
E = 64
F = 128
NC = 2
NS = 16
NW = NC * NS
BPW = B // NW
BG = 16

_sc_mesh = plsc.VectorSubcoreMesh(core_axis_name="c", subcore_axis_name="s")


@functools.partial(
    pl.kernel,
    mesh=_sc_mesh,
    compiler_params=pltpu.CompilerParams(use_tc_tiling_on_sc=False),
    out_type=(
        jax.ShapeDtypeStruct((B, E), jnp.float32),
        jax.ShapeDtypeStruct((B, BG), jnp.float32),
    ),
    scratch_types=[
        pltpu.VMEM((BPW,), jnp.int32),
        pltpu.VMEM((BPW,), jnp.int32),
        pltpu.VMEM((BPW, E), jnp.float32),
        pltpu.VMEM((BPW, BG), jnp.float32),
        pltpu.SemaphoreType.DMA,
        pltpu.SemaphoreType.DMA,
    ],
)
def _sc_gather(emb_hbm, bias_hbm, idx_hbm, idxg_hbm, emb_out, bias_out,
               idx_v, idxg_v, rows_v, brows_v, sem_e, sem_b):
    wid = lax.axis_index("s") * NC + lax.axis_index("c")
    base = wid * BPW
    pltpu.sync_copy(idx_hbm.at[pl.ds(base, BPW)], idx_v)
    pltpu.sync_copy(idxg_hbm.at[pl.ds(base, BPW)], idxg_v)
    ce = pltpu.async_copy(emb_hbm.at[idx_v], rows_v, sem_e)
    cb = pltpu.async_copy(bias_hbm.at[idxg_v], brows_v, sem_b)
    ce.wait()
    pltpu.sync_copy(rows_v, emb_out.at[pl.ds(base, BPW)])
    cb.wait()
    pltpu.sync_copy(brows_v, bias_out.at[pl.ds(base, BPW)])


BLK = 2048


def _tc_body(feats_ref, w_ref, b_ref, emb_ref, biasg_ref, ids_ref, out_ref):
    proj = jnp.dot(feats_ref[...], w_ref[...],
                   preferred_element_type=jnp.float32) + b_ref[...]
    d = jnp.sum(emb_ref[...] * proj, axis=1, keepdims=True)
    lane = ids_ref[...] & (BG - 1)
    sel = lax.broadcasted_iota(jnp.int32, (BLK, BG), 1) == lane
    bias = jnp.sum(jnp.where(sel, biasg_ref[...], 0.0), axis=1, keepdims=True)
    out_ref[...] = jax.nn.sigmoid(d + bias)


_tc_combine = pl.pallas_call(
    _tc_body,
    grid=(B // BLK,),
    in_specs=[
        pl.BlockSpec((BLK, F), lambda i: (i, 0)),
        pl.BlockSpec((F, E), lambda i: (0, 0)),
        pl.BlockSpec((1, E), lambda i: (0, 0)),
        pl.BlockSpec((BLK, E), lambda i: (i, 0)),
        pl.BlockSpec((BLK, BG), lambda i: (i, 0)),
        pl.BlockSpec((BLK, 1), lambda i: (i, 0)),
    ],
    out_specs=pl.BlockSpec((BLK, 1), lambda i: (i, 0)),
    out_shape=jax.ShapeDtypeStruct((B, 1), jnp.float32),
)


def kernel(user_ids, restaurant_features, user_emb, user_bias_table,
           dense_W, dense_b):
    ids = user_ids.astype(jnp.int32)
    idx = ids.reshape(B)
    idx_grp = idx >> 4
    bias2 = user_bias_table.reshape(-1, BG)
    emb_g, bias_g = _sc_gather(user_emb, bias2, idx, idx_grp)
    return _tc_combine(restaurant_features, dense_W, dense_b.reshape(1, E),
                       emb_g, bias_g, ids)


# speedup vs baseline: 1.8036x; 1.8036x over previous
import functools

import jax
import jax.numpy as jnp
from jax import lax
from jax.experimental import pallas as pl
from jax.experimental.pallas import tpu as pltpu
from jax.experimental.pallas import tpu_sc as plsc

B = 16384
E = 64
F = 128
U = 1000000
HALF = 524288
CB = 4096
NC = 2
NS = 16
NW = NC * NS
BPW = B // NW

def _xpose_body(lo_ref, hi_ref, out_ref):
    out_ref[:, :E] = lo_ref[...].T
    out_ref[:, E:] = hi_ref[...].T


_n_colblocks = (U + CB - 1) // CB

_tc_xpose = pl.pallas_call(
    _xpose_body,
    grid=(HALF // CB,),
    in_specs=[
        pl.BlockSpec((E, CB), lambda j: (0, j)),
        pl.BlockSpec((E, CB),
                     lambda j: (0, jnp.minimum(j + HALF // CB,
                                               _n_colblocks - 1))),
    ],
    out_specs=pl.BlockSpec((CB, 2 * E), lambda j: (j, 0)),
    out_shape=jax.ShapeDtypeStruct((HALF, 2 * E), jnp.float32),
)

_sc_mesh = plsc.VectorSubcoreMesh(core_axis_name="c", subcore_axis_name="s")


@functools.partial(
    pl.kernel,
    mesh=_sc_mesh,
    out_type=(
        jax.ShapeDtypeStruct((B, 2 * E), jnp.float32),
        jax.ShapeDtypeStruct((B,), jnp.float32),
    ),
    scratch_types=[
        pltpu.VMEM((BPW,), jnp.int32),
        pltpu.VMEM((BPW,), jnp.int32),
        pltpu.VMEM((BPW, 2 * E), jnp.float32),
        pltpu.VMEM((BPW,), jnp.float32),
        pltpu.SemaphoreType.DMA,
        pltpu.SemaphoreType.DMA,
    ],
)
def _sc_gather(p_hbm, bias_hbm, pidx_hbm, idx_hbm, rows_out, bias_out,
               pidx_v, idx_v, rows_v, bias_v, sem_e, sem_b):
    wid = lax.axis_index("s") * NC + lax.axis_index("c")
    base = wid * BPW
    pltpu.sync_copy(pidx_hbm.at[pl.ds(base, BPW)], pidx_v)
    pltpu.sync_copy(idx_hbm.at[pl.ds(base, BPW)], idx_v)
    ce = pltpu.async_copy(p_hbm.at[pidx_v], rows_v, sem_e)
    cb = pltpu.async_copy(bias_hbm.at[idx_v], bias_v, sem_b)
    ce.wait()
    pltpu.sync_copy(rows_v, rows_out.at[pl.ds(base, BPW)])
    cb.wait()
    pltpu.sync_copy(bias_v, bias_out.at[pl.ds(base, BPW)])


BLK = 2048


def _tc_body(feats_ref, w_ref, b_ref, rows_ref, bias_ref, ids_ref, out_ref):
    proj = jnp.dot(feats_ref[...], w_ref[...],
                   preferred_element_type=jnp.float32) + b_ref[...]
    hi = ids_ref[...] >= HALF
    emb = jnp.where(hi, rows_ref[:, E:], rows_ref[:, :E])
    d = jnp.sum(emb * proj, axis=1, keepdims=True)
    out_ref[...] = jax.nn.sigmoid(d + bias_ref[...])


_tc_combine = pl.pallas_call(
    _tc_body,
    grid=(B // BLK,),
    in_specs=[
        pl.BlockSpec((BLK, F), lambda i: (i, 0)),
        pl.BlockSpec((F, E), lambda i: (0, 0)),
        pl.BlockSpec((1, E), lambda i: (0, 0)),
        pl.BlockSpec((BLK, 2 * E), lambda i: (i, 0)),
        pl.BlockSpec((BLK, 1), lambda i: (i, 0)),
        pl.BlockSpec((BLK, 1), lambda i: (i, 0)),
    ],
    out_specs=pl.BlockSpec((BLK, 1), lambda i: (i, 0)),
    out_shape=jax.ShapeDtypeStruct((B, 1), jnp.float32),
)


def kernel(user_ids, restaurant_features, user_emb, user_bias_table,
           dense_W, dense_b):
    ids = user_ids.astype(jnp.int32)
    idx = ids.reshape(B)
    pidx = jnp.where(idx >= HALF, idx - HALF, idx)
    embt = user_emb.T
    bias_flat = user_bias_table.T.reshape(U)
    p = _tc_xpose(embt, embt)
    rows_g, bias_g = _sc_gather(p, bias_flat, pidx, idx)
    return _tc_combine(restaurant_features, dense_W, dense_b.reshape(1, E),
                       rows_g, bias_g.reshape(B, 1), ids)


# speedup vs baseline: 2.0789x; 1.1526x over previous
import functools

import jax
import jax.numpy as jnp
from jax import lax
from jax.experimental import pallas as pl
from jax.experimental.pallas import tpu as pltpu
from jax.experimental.pallas import tpu_sc as plsc

B = 16384
E = 64
F = 128
U = 1000000
HALF = 524288
CB = 8192
NC = 2
NS = 16
NW = NC * NS
BPW = B // NW

def _xpose_body(lo_ref, hi_ref, out_ref):
    out_ref[:, :E] = lo_ref[...].T
    out_ref[:, E:] = hi_ref[...].T


_n_colblocks = (U + CB - 1) // CB

_tc_xpose = pl.pallas_call(
    _xpose_body,
    grid=(HALF // CB,),
    in_specs=[
        pl.BlockSpec((E, CB), lambda j: (0, j)),
        pl.BlockSpec((E, CB),
                     lambda j: (0, jnp.minimum(j + HALF // CB,
                                               _n_colblocks - 1))),
    ],
    out_specs=pl.BlockSpec((CB, 2 * E), lambda j: (j, 0)),
    out_shape=jax.ShapeDtypeStruct((HALF, 2 * E), jnp.float32),
)

_sc_mesh = plsc.VectorSubcoreMesh(core_axis_name="c", subcore_axis_name="s")


@functools.partial(
    pl.kernel,
    mesh=_sc_mesh,
    out_type=(
        jax.ShapeDtypeStruct((B, 2 * E), jnp.float32),
        jax.ShapeDtypeStruct((B,), jnp.float32),
    ),
    scratch_types=[
        pltpu.VMEM((BPW,), jnp.int32),
        pltpu.VMEM((BPW,), jnp.int32),
        pltpu.VMEM((BPW, 2 * E), jnp.float32),
        pltpu.VMEM((BPW,), jnp.float32),
        pltpu.SemaphoreType.DMA,
        pltpu.SemaphoreType.DMA,
    ],
)
def _sc_gather(p_hbm, bias_hbm, pidx_hbm, idx_hbm, rows_out, bias_out,
               pidx_v, idx_v, rows_v, bias_v, sem_e, sem_b):
    wid = lax.axis_index("s") * NC + lax.axis_index("c")
    base = wid * BPW
    pltpu.sync_copy(pidx_hbm.at[pl.ds(base, BPW)], pidx_v)
    pltpu.sync_copy(idx_hbm.at[pl.ds(base, BPW)], idx_v)
    ce = pltpu.async_copy(p_hbm.at[pidx_v], rows_v, sem_e)
    cb = pltpu.async_copy(bias_hbm.at[idx_v], bias_v, sem_b)
    ce.wait()
    pltpu.sync_copy(rows_v, rows_out.at[pl.ds(base, BPW)])
    cb.wait()
    pltpu.sync_copy(bias_v, bias_out.at[pl.ds(base, BPW)])


BLK = 2048


def _tc_body(feats_ref, w_ref, b_ref, rows_ref, bias_ref, idx_ref, out_ref):
    proj = jnp.dot(feats_ref[...], w_ref[...],
                   preferred_element_type=jnp.float32) + b_ref[...]
    d_lo = jnp.sum(rows_ref[:, :E] * proj, axis=1, keepdims=True)
    d_hi = jnp.sum(rows_ref[:, E:] * proj, axis=1, keepdims=True)
    hi = idx_ref[...] >= HALF
    d = jnp.where(hi, d_hi.T, d_lo.T)
    out_ref[...] = jax.nn.sigmoid(d + bias_ref[...])


_tc_combine = pl.pallas_call(
    _tc_body,
    grid=(B // BLK,),
    in_specs=[
        pl.BlockSpec((BLK, F), lambda i: (i, 0)),
        pl.BlockSpec((F, E), lambda i: (0, 0)),
        pl.BlockSpec((1, E), lambda i: (0, 0)),
        pl.BlockSpec((BLK, 2 * E), lambda i: (i, 0)),
        pl.BlockSpec((1, BLK), lambda i: (0, i)),
        pl.BlockSpec((1, BLK), lambda i: (0, i)),
    ],
    out_specs=pl.BlockSpec((1, BLK), lambda i: (0, i)),
    out_shape=jax.ShapeDtypeStruct((1, B), jnp.float32),
)


def kernel(user_ids, restaurant_features, user_emb, user_bias_table,
           dense_W, dense_b):
    idx = user_ids.astype(jnp.int32).reshape(B)
    pidx = jnp.where(idx >= HALF, idx - HALF, idx)
    embt = user_emb.T
    bias_flat = user_bias_table.T.reshape(U)
    p = _tc_xpose(embt, embt)
    rows_g, bias_g = _sc_gather(p, bias_flat, pidx, idx)
    out = _tc_combine(restaurant_features, dense_W, dense_b.reshape(1, E),
                      rows_g, bias_g.reshape(1, B), idx.reshape(1, B))
    return out.reshape(B, 1)


# speedup vs baseline: 2.8017x; 1.3477x over previous
import functools

import jax
import jax.numpy as jnp
from jax import lax
from jax.experimental import pallas as pl
from jax.experimental.pallas import tpu as pltpu
from jax.experimental.pallas import tpu_sc as plsc

B = 16384
E = 64
F = 128
U = 1000000
Q = 262144
CB = 8192
NC = 2
NS = 16
NW = NC * NS
BPW = B // NW

_n_colblocks = (U + CB - 1) // CB
_QB = Q // CB


def _pack_pair(lo_f32, hi_f32):
    lo16 = lax.bitcast_convert_type(lo_f32.astype(jnp.bfloat16), jnp.uint16)
    hi16 = lax.bitcast_convert_type(hi_f32.astype(jnp.bfloat16), jnp.uint16)
    word = lo16.astype(jnp.uint32) | (hi16.astype(jnp.uint32) << 16)
    return lax.bitcast_convert_type(word, jnp.float32)


def _xpose_body(c0_ref, c1_ref, c2_ref, c3_ref, out_ref):
    i0 = lax.broadcasted_iota(jnp.int32, (E, E), 0)
    i1 = lax.broadcasted_iota(jnp.int32, (E, E), 1)
    ident = jnp.where(i0 == i1, 1.0, 0.0).astype(jnp.bfloat16)
    dn = (((0,), (0,)), ((), ()))

    def t(ref):
        return lax.dot_general(ref[...].astype(jnp.bfloat16), ident, dn,
                               preferred_element_type=jnp.float32)

    out_ref[:, :E] = _pack_pair(t(c0_ref), t(c1_ref))
    out_ref[:, E:] = _pack_pair(t(c2_ref), t(c3_ref))


def _in_spec(q):
    return pl.BlockSpec(
        (E, CB),
        lambda j, q=q: (0, jnp.minimum(j + q * _QB, _n_colblocks - 1)))


_tc_xpose = pl.pallas_call(
    _xpose_body,
    grid=(_QB,),
    in_specs=[_in_spec(0), _in_spec(1), _in_spec(2), _in_spec(3)],
    out_specs=pl.BlockSpec((CB, 2 * E), lambda j: (j, 0)),
    out_shape=jax.ShapeDtypeStruct((Q, 2 * E), jnp.float32),
)

_sc_mesh = plsc.VectorSubcoreMesh(core_axis_name="c", subcore_axis_name="s")


@functools.partial(
    pl.kernel,
    mesh=_sc_mesh,
    out_type=(
        jax.ShapeDtypeStruct((B, 2 * E), jnp.float32),
        jax.ShapeDtypeStruct((B,), jnp.float32),
    ),
    scratch_types=[
        pltpu.VMEM((BPW,), jnp.int32),
        pltpu.VMEM((BPW,), jnp.int32),
        pltpu.VMEM((BPW, 2 * E), jnp.float32),
        pltpu.VMEM((BPW,), jnp.float32),
        pltpu.SemaphoreType.DMA,
        pltpu.SemaphoreType.DMA,
    ],
)
def _sc_gather(p_hbm, bias_hbm, pidx_hbm, idx_hbm, rows_out, bias_out,
               pidx_v, idx_v, rows_v, bias_v, sem_e, sem_b):
    wid = lax.axis_index("s") * NC + lax.axis_index("c")
    base = wid * BPW
    pltpu.sync_copy(pidx_hbm.at[pl.ds(base, BPW)], pidx_v)
    pltpu.sync_copy(idx_hbm.at[pl.ds(base, BPW)], idx_v)
    ce = pltpu.async_copy(p_hbm.at[pidx_v], rows_v, sem_e)
    cb = pltpu.async_copy(bias_hbm.at[idx_v], bias_v, sem_b)
    ce.wait()
    pltpu.sync_copy(rows_v, rows_out.at[pl.ds(base, BPW)])
    cb.wait()
    pltpu.sync_copy(bias_v, bias_out.at[pl.ds(base, BPW)])


BLK = 2048


def _unpack(word_f32, hi):
    u = lax.bitcast_convert_type(word_f32, jnp.uint32)
    h = jnp.where(hi, u >> 16, u & 0xFFFF).astype(jnp.uint16)
    return lax.bitcast_convert_type(h, jnp.bfloat16).astype(jnp.float32)


def _tc_body(feats_ref, w_ref, b_ref, rows_ref, bias_ref, idx_ref, out_ref):
    proj = jnp.dot(feats_ref[...], w_ref[...],
                   preferred_element_type=jnp.float32) + b_ref[...]
    d = []
    for half in (rows_ref[:, :E], rows_ref[:, E:]):
        for hi in (False, True):
            emb = _unpack(half, hi)
            d.append(jnp.sum(emb * proj, axis=1, keepdims=True).T)
    quarter = idx_ref[...] // Q
    dq = jnp.where(quarter >= 2,
                   jnp.where(quarter == 3, d[3], d[2]),
                   jnp.where(quarter == 1, d[1], d[0]))
    out_ref[...] = jax.nn.sigmoid(dq + bias_ref[...])


_tc_combine = pl.pallas_call(
    _tc_body,
    grid=(B // BLK,),
    in_specs=[
        pl.BlockSpec((BLK, F), lambda i: (i, 0)),
        pl.BlockSpec((F, E), lambda i: (0, 0)),
        pl.BlockSpec((1, E), lambda i: (0, 0)),
        pl.BlockSpec((BLK, 2 * E), lambda i: (i, 0)),
        pl.BlockSpec((1, BLK), lambda i: (0, i)),
        pl.BlockSpec((1, BLK), lambda i: (0, i)),
    ],
    out_specs=pl.BlockSpec((1, BLK), lambda i: (0, i)),
    out_shape=jax.ShapeDtypeStruct((1, B), jnp.float32),
)


def kernel(user_ids, restaurant_features, user_emb, user_bias_table,
           dense_W, dense_b):
    idx = user_ids.astype(jnp.int32).reshape(B)
    pidx = idx % Q
    embt = user_emb.T
    bias_flat = user_bias_table.T.reshape(U)
    p = _tc_xpose(embt, embt, embt, embt)
    rows_g, bias_g = _sc_gather(p, bias_flat, pidx, idx)
    out = _tc_combine(restaurant_features, dense_W, dense_b.reshape(1, E),
                      rows_g, bias_g.reshape(1, B), idx.reshape(1, B))
    return out.reshape(B, 1)


# speedup vs baseline: 2.8988x; 1.0347x over previous
import functools

import jax
import jax.numpy as jnp
from jax import lax
from jax.experimental import pallas as pl
from jax.experimental.pallas import tpu as pltpu
from jax.experimental.pallas import tpu_sc as plsc

B = 16384
E = 64
F = 128
U = 1000000
Q = 262144
CB = 16384
NC = 2
NS = 16
NW = NC * NS
BPW = B // NW

_n_colblocks = (U + CB - 1) // CB
_QB = Q // CB


def _pack_pair(lo_f32, hi_f32):
    lo16 = lax.bitcast_convert_type(lo_f32.astype(jnp.bfloat16), jnp.uint16)
    hi16 = lax.bitcast_convert_type(hi_f32.astype(jnp.bfloat16), jnp.uint16)
    word = lo16.astype(jnp.uint32) | (hi16.astype(jnp.uint32) << 16)
    return lax.bitcast_convert_type(word, jnp.float32)


def _xpose_body(c0_ref, c1_ref, c2_ref, c3_ref, out_ref):
    i0 = lax.broadcasted_iota(jnp.int32, (E, E), 0)
    i1 = lax.broadcasted_iota(jnp.int32, (E, E), 1)
    ident = jnp.where(i0 == i1, 1.0, 0.0).astype(jnp.bfloat16)
    dn = (((0,), (0,)), ((), ()))

    def t(ref):
        return lax.dot_general(ref[...].astype(jnp.bfloat16), ident, dn,
                               preferred_element_type=jnp.float32)

    out_ref[:, :E] = _pack_pair(t(c0_ref), t(c1_ref))
    out_ref[:, E:] = _pack_pair(t(c2_ref), t(c3_ref))


def _in_spec(q):
    return pl.BlockSpec(
        (E, CB),
        lambda j, q=q: (0, jnp.minimum(j + q * _QB, _n_colblocks - 1)))


_tc_xpose = pl.pallas_call(
    _xpose_body,
    grid=(_QB,),
    in_specs=[_in_spec(0), _in_spec(1), _in_spec(2), _in_spec(3)],
    out_specs=pl.BlockSpec((CB, 2 * E), lambda j: (j, 0)),
    out_shape=jax.ShapeDtypeStruct((Q, 2 * E), jnp.float32),
)

_sc_mesh = plsc.VectorSubcoreMesh(core_axis_name="c", subcore_axis_name="s")


@functools.partial(
    pl.kernel,
    mesh=_sc_mesh,
    out_type=(
        jax.ShapeDtypeStruct((B, 2 * E), jnp.float32),
        jax.ShapeDtypeStruct((B,), jnp.float32),
    ),
    scratch_types=[
        pltpu.VMEM((BPW,), jnp.int32),
        pltpu.VMEM((BPW,), jnp.int32),
        pltpu.VMEM((BPW, 2 * E), jnp.float32),
        pltpu.VMEM((BPW,), jnp.float32),
        pltpu.SemaphoreType.DMA,
        pltpu.SemaphoreType.DMA,
    ],
)
def _sc_gather(p_hbm, bias_hbm, pidx_hbm, idx_hbm, rows_out, bias_out,
               pidx_v, idx_v, rows_v, bias_v, sem_e, sem_b):
    wid = lax.axis_index("s") * NC + lax.axis_index("c")
    base = wid * BPW
    pltpu.sync_copy(pidx_hbm.at[pl.ds(base, BPW)], pidx_v)
    pltpu.sync_copy(idx_hbm.at[pl.ds(base, BPW)], idx_v)
    ce = pltpu.async_copy(p_hbm.at[pidx_v], rows_v, sem_e)
    cb = pltpu.async_copy(bias_hbm.at[idx_v], bias_v, sem_b)
    ce.wait()
    pltpu.sync_copy(rows_v, rows_out.at[pl.ds(base, BPW)])
    cb.wait()
    pltpu.sync_copy(bias_v, bias_out.at[pl.ds(base, BPW)])


BLK = 4096


def _unpack(word_f32, hi):
    u = lax.bitcast_convert_type(word_f32, jnp.uint32)
    h = jnp.where(hi, u >> 16, u & 0xFFFF).astype(jnp.uint16)
    return lax.bitcast_convert_type(h, jnp.bfloat16).astype(jnp.float32)


def _tc_body(feats_ref, w_ref, b_ref, rows_ref, bias_ref, idx_ref, out_ref):
    proj = jnp.dot(feats_ref[...], w_ref[...],
                   preferred_element_type=jnp.float32) + b_ref[...]
    d = []
    for half in (rows_ref[:, :E], rows_ref[:, E:]):
        for hi in (False, True):
            emb = _unpack(half, hi)
            d.append(jnp.sum(emb * proj, axis=1, keepdims=True).T)
    quarter = idx_ref[...] // Q
    dq = jnp.where(quarter >= 2,
                   jnp.where(quarter == 3, d[3], d[2]),
                   jnp.where(quarter == 1, d[1], d[0]))
    out_ref[...] = jax.nn.sigmoid(dq + bias_ref[...])


_tc_combine = pl.pallas_call(
    _tc_body,
    grid=(B // BLK,),
    in_specs=[
        pl.BlockSpec((BLK, F), lambda i: (i, 0)),
        pl.BlockSpec((F, E), lambda i: (0, 0)),
        pl.BlockSpec((1, E), lambda i: (0, 0)),
        pl.BlockSpec((BLK, 2 * E), lambda i: (i, 0)),
        pl.BlockSpec((1, BLK), lambda i: (0, i)),
        pl.BlockSpec((1, BLK), lambda i: (0, i)),
    ],
    out_specs=pl.BlockSpec((1, BLK), lambda i: (0, i)),
    out_shape=jax.ShapeDtypeStruct((1, B), jnp.float32),
)


def kernel(user_ids, restaurant_features, user_emb, user_bias_table,
           dense_W, dense_b):
    idx = user_ids.astype(jnp.int32).reshape(B)
    pidx = idx % Q
    embt = user_emb.T
    bias_flat = user_bias_table.T.reshape(U)
    p = _tc_xpose(embt, embt, embt, embt)
    rows_g, bias_g = _sc_gather(p, bias_flat, pidx, idx)
    out = _tc_combine(restaurant_features, dense_W, dense_b.reshape(1, E),
                      rows_g, bias_g.reshape(1, B), idx.reshape(1, B))
    return out.reshape(B, 1)
